# baseline (speedup 1.0000x reference)
import functools

import jax
import jax.numpy as jnp
from jax import lax
from jax.experimental import pallas as pl
from jax.experimental.pallas import tpu as pltpu
from jax.experimental.pallas import tpu_sc as plsc


def _project_table(table, W, b2):
    V, D = table.shape
    F = W.shape[1]
    R = 10000

    def body(t_ref, w_ref, b_ref, o_ref):
        acc = jnp.dot(
            t_ref[...].astype(jnp.bfloat16),
            w_ref[...].astype(jnp.bfloat16),
            preferred_element_type=jnp.float32,
        )
        o_ref[...] = jnp.tanh(acc + b_ref[...])

    return pl.pallas_call(
        body,
        grid=(V // R,),
        in_specs=[
            pl.BlockSpec((R, D), lambda i: (i, 0)),
            pl.BlockSpec((D, F), lambda i: (0, 0)),
            pl.BlockSpec((1, F), lambda i: (0, 0)),
        ],
        out_specs=pl.BlockSpec((R, F), lambda i: (i, 0)),
        out_shape=jax.ShapeDtypeStruct((V, F), jnp.float32),
    )(table, W, b2)


_NC = 2
_NS = 16
_NW = _NC * _NS
_CH = 400


def _sc_gather(q, idx_flat):
    n = idx_flat.shape[0]
    F = q.shape[1]
    b_per_w = n // _NW
    n_ch = b_per_w // _CH
    mesh = plsc.VectorSubcoreMesh(core_axis_name="c", subcore_axis_name="s")

    @functools.partial(
        pl.kernel,
        mesh=mesh,
        out_type=jax.ShapeDtypeStruct((n, F), q.dtype),
        scratch_types=(
            [pltpu.VMEM((b_per_w,), jnp.int32)]
            + [pltpu.VMEM((_CH, F), jnp.float32) for _ in range(2)]
            + [pltpu.SemaphoreType.DMA for _ in range(4)]
        ),
    )
    def k(q_hbm, i_hbm, o_hbm, idx_v, b0, b1, g0, g1, s0, s1):
        bufs = (b0, b1)
        gsem = (g0, g1)
        osem = (s0, s1)
        wid = lax.axis_index("s") * _NC + lax.axis_index("c")
        base = wid * b_per_w
        pltpu.sync_copy(i_hbm.at[pl.ds(base, b_per_w)], idx_v)

        def gather_start(c, j):
            pltpu.async_copy(q_hbm.at[idx_v.at[pl.ds(c * _CH, _CH)]], bufs[j], gsem[j])

        def gather_wait(j):
            pltpu.make_async_copy(
                q_hbm.at[idx_v.at[pl.ds(0, _CH)]], bufs[j], gsem[j]
            ).wait()

        def out_start(c, j):
            pltpu.async_copy(bufs[j], o_hbm.at[pl.ds(base + c * _CH, _CH)], osem[j])

        def out_wait(j):
            pltpu.make_async_copy(
                bufs[j], o_hbm.at[pl.ds(base, _CH)], osem[j]
            ).wait()

        gather_start(0, 0)
        gather_start(1, 1)
        gather_wait(0)
        out_start(0, 0)

        @pl.loop(1, n_ch - 1, step=2)
        def _(c0):
            for j2 in range(2):
                i = c0 + j2
                j = (1 + j2) % 2
                jn = j2
                out_wait(jn)
                gather_start(i + 1, jn)
                gather_wait(j)
                out_start(i, j)

        gather_wait(1)
        out_start(n_ch - 1, 1)
        out_wait(0)
        out_wait(1)

    return k(q, idx_flat)


def kernel(inputs, table, W, b):
    Bsz, L = inputs.shape
    F = W.shape[1]
    q = _project_table(table, W, b.reshape(1, F))
    flat = inputs.reshape(Bsz * L).astype(jnp.int32)
    out = _sc_gather(q, flat)
    return out.reshape(Bsz, L, F)


# speedup vs baseline: 1.0081x; 1.0081x over previous
import functools

import jax
import jax.numpy as jnp
from jax import lax
from jax.experimental import pallas as pl
from jax.experimental.pallas import tpu as pltpu
from jax.experimental.pallas import tpu_sc as plsc


def _project_table(table, W, b2):
    V, D = table.shape
    F = W.shape[1]
    R = 20000

    def body(t_ref, w_ref, b_ref, o_ref):
        acc = jnp.dot(t_ref[...], w_ref[...], preferred_element_type=jnp.float32)
        o_ref[...] = jnp.tanh(acc + b_ref[...])

    return pl.pallas_call(
        body,
        grid=(V // R,),
        in_specs=[
            pl.BlockSpec((R, D), lambda i: (i, 0)),
            pl.BlockSpec((D, F), lambda i: (0, 0)),
            pl.BlockSpec((1, F), lambda i: (0, 0)),
        ],
        out_specs=pl.BlockSpec((R, F), lambda i: (i, 0)),
        out_shape=jax.ShapeDtypeStruct((V, F), jnp.float32),
    )(table, W, b2)


_NC = 2
_NS = 16
_NW = _NC * _NS
_CH = 400


def _sc_gather(q, idx_flat):
    n = idx_flat.shape[0]
    F = q.shape[1]
    b_per_w = n // _NW
    n_ch = b_per_w // _CH
    mesh = plsc.VectorSubcoreMesh(core_axis_name="c", subcore_axis_name="s")

    @functools.partial(
        pl.kernel,
        mesh=mesh,
        out_type=jax.ShapeDtypeStruct((n, F), q.dtype),
        scratch_types=(
            [pltpu.VMEM((b_per_w,), jnp.int32)]
            + [pltpu.VMEM((_CH, F), jnp.float32) for _ in range(2)]
            + [pltpu.SemaphoreType.DMA for _ in range(4)]
        ),
    )
    def k(q_hbm, i_hbm, o_hbm, idx_v, b0, b1, g0, g1, s0, s1):
        bufs = (b0, b1)
        gsem = (g0, g1)
        osem = (s0, s1)
        wid = lax.axis_index("s") * _NC + lax.axis_index("c")
        base = wid * b_per_w
        pltpu.sync_copy(i_hbm.at[pl.ds(base, b_per_w)], idx_v)

        def gather_start(c, j):
            pltpu.async_copy(q_hbm.at[idx_v.at[pl.ds(c * _CH, _CH)]], bufs[j], gsem[j])

        def gather_wait(j):
            pltpu.make_async_copy(
                q_hbm.at[idx_v.at[pl.ds(0, _CH)]], bufs[j], gsem[j]
            ).wait()

        def out_start(c, j):
            pltpu.async_copy(bufs[j], o_hbm.at[pl.ds(base + c * _CH, _CH)], osem[j])

        def out_wait(j):
            pltpu.make_async_copy(
                bufs[j], o_hbm.at[pl.ds(base, _CH)], osem[j]
            ).wait()

        gather_start(0, 0)
        gather_start(1, 1)
        gather_wait(0)
        out_start(0, 0)

        @pl.loop(1, n_ch - 1, step=2)
        def _(c0):
            for j2 in range(2):
                i = c0 + j2
                j = (1 + j2) % 2
                jn = j2
                out_wait(jn)
                gather_start(i + 1, jn)
                gather_wait(j)
                out_start(i, j)

        gather_wait(1)
        out_start(n_ch - 1, 1)
        out_wait(0)
        out_wait(1)

    return k(q, idx_flat)


def kernel(inputs, table, W, b):
    Bsz, L = inputs.shape
    F = W.shape[1]
    q = _project_table(table, W, b.reshape(1, F))
    flat = inputs.reshape(Bsz * L).astype(jnp.int32)
    out = _sc_gather(q, flat)
    return out.reshape(Bsz, L, F)
